# Initial kernel scaffold; baseline (speedup 1.0000x reference)
import jax
import jax.numpy as jnp
from jax.experimental import pallas as pl


def kernel(x, xk, delta_raw, scale_raw, shift):
    raise NotImplementedError("write your pallas kernel here")


# speedup vs baseline: 2434.8444x; 2434.8444x over previous
import functools

import jax
import jax.numpy as jnp
from jax import lax
from jax.experimental import pallas as pl

D = 256
K = 64
BN = 512


def _prep_kernel(xkT_ref, dpT_ref, sc_ref, sh_ref, xkc_ref, dA_ref, dB_ref):
    f32 = jnp.float32
    xkT = xkT_ref[...]
    dpT = dpT_ref[...]
    row = lax.broadcasted_iota(jnp.int32, (K, K), 0)
    col = lax.broadcasted_iota(jnp.int32, (K, K), 1)
    rmask = lax.broadcasted_iota(jnp.int32, (K, D), 0)

    m1 = (col == row + 1).astype(f32) - (col == row).astype(f32)
    dxT = jnp.dot(m1, xkT, preferred_element_type=f32)
    dxT = jnp.where(rmask == K - 1, 0.0, dxT)

    sT = jax.nn.softplus(dpT) + 1e-4
    sdx = sT * dxT
    avg = jnp.sum(sdx, axis=0, keepdims=True) / (
        jnp.sum(dxT, axis=0, keepdims=True) + 1e-8)
    sT = sT / (avg + 1e-8)
    sdx = sT * dxT

    tm = (col < row).astype(f32)
    ykT = jnp.dot(tm, sdx, preferred_element_type=f32)

    scale = jax.nn.softplus(sc_ref[...]) + 1e-3
    shiftv = sh_ref[...]
    aT = sT * scale
    bT = (ykT - sT * xkT) * scale + shiftv

    d2 = (col == row).astype(f32) - (col == row - 1).astype(f32)
    dA = jnp.dot(d2, aT, preferred_element_type=f32)
    dB = jnp.dot(d2, bT, preferred_element_type=f32)
    last = rmask == K - 1
    dA_ref[...] = jnp.where(last, 0.0, dA)
    dB_ref[...] = jnp.where(last, 0.0, dB)
    xkc_ref[...] = jnp.where(rmask == 0, -jnp.inf, xkT)


def _scan_kernel(x_ref, xkc_ref, dA_ref, dB_ref, o_ref):
    x = x_ref[...]
    xkc = xkc_ref[...]
    dA = dA_ref[...]
    dB = dB_ref[...]
    a = jnp.zeros_like(x)
    b = jnp.zeros_like(x)
    for j in range(K):
        f = (x > xkc[j, :][None, :]).astype(jnp.float32)
        a = a + f * dA[j, :][None, :]
        b = b + f * dB[j, :][None, :]
    o_ref[...] = a * x + b


@functools.partial(jax.jit, static_argnames=())
def kernel(x, xk, delta_raw, scale_raw, shift):
    f32 = jnp.float32
    n = x.shape[0]
    xkT = xk.T.astype(f32)
    dpT = jnp.pad(delta_raw, ((0, 0), (0, 1))).T.astype(f32)
    sc = scale_raw[None, :].astype(f32)
    sh = shift[None, :].astype(f32)

    xkc, dA, dB = pl.pallas_call(
        _prep_kernel,
        out_shape=[jax.ShapeDtypeStruct((K, D), f32)] * 3,
    )(xkT, dpT, sc, sh)

    tab_spec = pl.BlockSpec((K, D), lambda i: (0, 0))
    out = pl.pallas_call(
        _scan_kernel,
        grid=(n // BN,),
        in_specs=[pl.BlockSpec((BN, D), lambda i: (i, 0)),
                  tab_spec, tab_spec, tab_spec],
        out_specs=pl.BlockSpec((BN, D), lambda i: (i, 0)),
        out_shape=jax.ShapeDtypeStruct((n, D), f32),
    )(x, xkc, dA, dB)
    return out


# speedup vs baseline: 2810.1305x; 1.1541x over previous
import functools

import jax
import jax.numpy as jnp
from jax import lax
from jax.experimental import pallas as pl

D = 256
K = 64
BN = 512


def _prep_kernel(xkT_ref, dpT_ref, sc_ref, sh_ref, xkc_ref, dA_ref, dB_ref):
    f32 = jnp.float32
    xkT = xkT_ref[...]
    dpT = dpT_ref[...]
    row = lax.broadcasted_iota(jnp.int32, (K, K), 0)
    col = lax.broadcasted_iota(jnp.int32, (K, K), 1)
    rmask = lax.broadcasted_iota(jnp.int32, (K, D), 0)

    m1 = (col == row + 1).astype(f32) - (col == row).astype(f32)
    dxT = jnp.dot(m1, xkT, preferred_element_type=f32)
    dxT = jnp.where(rmask == K - 1, 0.0, dxT)

    sT = jax.nn.softplus(dpT) + 1e-4
    sdx = sT * dxT
    avg = jnp.sum(sdx, axis=0, keepdims=True) / (
        jnp.sum(dxT, axis=0, keepdims=True) + 1e-8)
    sT = sT / (avg + 1e-8)
    sdx = sT * dxT

    tm = (col < row).astype(f32)
    ykT = jnp.dot(tm, sdx, preferred_element_type=f32)

    scale = jax.nn.softplus(sc_ref[...]) + 1e-3
    shiftv = sh_ref[...]
    aT = sT * scale
    bT = (ykT - sT * xkT) * scale + shiftv

    d2 = (col == row).astype(f32) - (col == row - 1).astype(f32)
    dA = jnp.dot(d2, aT, preferred_element_type=f32)
    dB = jnp.dot(d2, bT, preferred_element_type=f32)
    last = rmask == K - 1
    dA_ref[...] = jnp.where(last, 0.0, dA)
    dB_ref[...] = jnp.where(last, 0.0, dB)
    xkc_ref[...] = jnp.where(rmask == 0, -jnp.inf, xkT)


def _scan_kernel(x_ref, xkc_ref, dA_ref, dB_ref, o_ref):
    x = x_ref[...]
    xkc = xkc_ref[...]
    dA = dA_ref[...]
    dB = dB_ref[...]
    a = jnp.zeros_like(x)
    b = jnp.zeros_like(x)
    for j in range(K):
        m = x > xkc[j, :][None, :]
        a = jnp.where(m, a + dA[j, :][None, :], a)
        b = jnp.where(m, b + dB[j, :][None, :], b)
    o_ref[...] = a * x + b


@functools.partial(jax.jit, static_argnames=())
def kernel(x, xk, delta_raw, scale_raw, shift):
    f32 = jnp.float32
    n = x.shape[0]
    xkT = xk.T.astype(f32)
    dpT = jnp.pad(delta_raw, ((0, 0), (0, 1))).T.astype(f32)
    sc = scale_raw[None, :].astype(f32)
    sh = shift[None, :].astype(f32)

    xkc, dA, dB = pl.pallas_call(
        _prep_kernel,
        out_shape=[jax.ShapeDtypeStruct((K, D), f32)] * 3,
    )(xkT, dpT, sc, sh)

    tab_spec = pl.BlockSpec((K, D), lambda i: (0, 0))
    out = pl.pallas_call(
        _scan_kernel,
        grid=(n // BN,),
        in_specs=[pl.BlockSpec((BN, D), lambda i: (i, 0)),
                  tab_spec, tab_spec, tab_spec],
        out_specs=pl.BlockSpec((BN, D), lambda i: (i, 0)),
        out_shape=jax.ShapeDtypeStruct((n, D), f32),
    )(x, xkc, dA, dB)
    return out


# speedup vs baseline: 4543.2094x; 1.6167x over previous
import jax
import jax.numpy as jnp
from jax import lax
from jax.experimental import pallas as pl

D = 256
K = 64
BN = 512


def _prep_kernel(xkT_ref, dpT_ref, sc_ref, sh_ref, xkt_ref, wt_ref):
    f32 = jnp.float32
    xkT = xkT_ref[...]
    dpT = dpT_ref[...]
    row = lax.broadcasted_iota(jnp.int32, (K, K), 0)
    col = lax.broadcasted_iota(jnp.int32, (K, K), 1)
    rmask = lax.broadcasted_iota(jnp.int32, (K, D), 0)

    m1 = (col == row + 1).astype(f32) - (col == row).astype(f32)
    dxT = jnp.dot(m1, xkT, preferred_element_type=f32)
    dxT = jnp.where(rmask == K - 1, 0.0, dxT)

    sT = jax.nn.softplus(dpT) + 1e-4
    avg = jnp.sum(sT * dxT, axis=0, keepdims=True) / (
        jnp.sum(dxT, axis=0, keepdims=True) + 1e-8)
    sT = sT / (avg + 1e-8)

    scale = jax.nn.softplus(sc_ref[...]) + 1e-3
    shiftv = sh_ref[...]

    d2 = (col == row).astype(f32) - (col == row - 1).astype(f32)
    ds = jnp.dot(d2, sT, preferred_element_type=f32)
    w = -scale * ds

    s0 = jnp.sum(jnp.where(rmask == 0, sT, 0.0), axis=0, keepdims=True)
    s_last = jnp.sum(jnp.where(rmask == K - 2, sT, 0.0), axis=0, keepdims=True)
    xk0 = jnp.sum(jnp.where(rmask == 0, xkT, 0.0), axis=0, keepdims=True)
    c1 = scale * s_last
    c0 = shiftv - scale * s0 * xk0

    wt = jnp.where(rmask == 0, c1, jnp.where(rmask == K - 1, c0, w))
    wt_ref[...] = wt
    xkt_ref[...] = xkT


def _scan_kernel(x_ref, xkt_ref, wt_ref, o_ref):
    x = x_ref[...]
    xkt = xkt_ref[...]
    wt = wt_ref[...]
    acc = x * wt[0, :][None, :] + wt[K - 1, :][None, :]
    for j in range(1, K - 1):
        acc = acc + wt[j, :][None, :] * jnp.minimum(x, xkt[j, :][None, :])
    o_ref[...] = acc


def kernel(x, xk, delta_raw, scale_raw, shift):
    f32 = jnp.float32
    n = x.shape[0]
    xkT = xk.T.astype(f32)
    dpT = jnp.pad(delta_raw, ((0, 0), (0, 1))).T.astype(f32)
    sc = scale_raw[None, :].astype(f32)
    sh = shift[None, :].astype(f32)

    xkt, wt = pl.pallas_call(
        _prep_kernel,
        out_shape=[jax.ShapeDtypeStruct((K, D), f32)] * 2,
    )(xkT, dpT, sc, sh)

    tab_spec = pl.BlockSpec((K, D), lambda i: (0, 0))
    out = pl.pallas_call(
        _scan_kernel,
        grid=(n // BN,),
        in_specs=[pl.BlockSpec((BN, D), lambda i: (i, 0)),
                  tab_spec, tab_spec],
        out_specs=pl.BlockSpec((BN, D), lambda i: (i, 0)),
        out_shape=jax.ShapeDtypeStruct((n, D), f32),
    )(x, xkt, wt)
    return out
